# baseline (speedup 1.0000x reference)
import numpy as np
import jax
import jax.numpy as jnp
from jax import lax
from jax.experimental import pallas as pl
from jax.experimental.pallas import tpu as pltpu
from jax.experimental.pallas import tpu_sc as plsc

_B, _N = 4096, 128
_K = _N * (_N + 1) // 2
_NT = _K // 16

_NW = 32
_BPW = _B // _NW
_GPW = _BPW // 8

_ROW_IDX, _COL_IDX = np.triu_indices(_N)
_RID = _ROW_IDX.astype(np.int32)
_CID = _COL_IDX.astype(np.int32)

_SEG_OFF = np.concatenate([[0], np.cumsum(np.arange(_N, 0, -1))])
_TILE_PLAN = []
for _t in range(_NT):
    _lo = 16 * _t
    _i = int(np.searchsorted(_SEG_OFF, _lo, side="right") - 1)
    if _SEG_OFF[_i + 1] >= _lo + 16:
        _TILE_PLAN.append((_lo, (_i, _i + (_lo - int(_SEG_OFF[_i])))))
    else:
        _TILE_PLAN.append((_lo, None))


def _sc_body(x_hbm, rid_hbm, cid_hbm, out_hbm, rid_v, cid_v, in_v, out_v,
             isem, osem):
    c = lax.axis_index("c")
    s = lax.axis_index("s")
    wid = s * 2 + c
    b0 = wid * _BPW

    pltpu.sync_copy(rid_hbm, rid_v)
    pltpu.sync_copy(cid_hbm, cid_v)

    def start_in(p, b):
        pltpu.async_copy(x_hbm.at[b], in_v.at[p], isem)

    def wait_in(p, b):
        pltpu.make_async_copy(x_hbm.at[b], in_v.at[p], isem).wait()

    def start_out(g):
        pltpu.async_copy(out_v, out_hbm.at[pl.ds(b0 + g * 8, 8)], osem)

    def wait_out(g):
        pltpu.make_async_copy(out_v, out_hbm.at[pl.ds(b0 + g * 8, 8)],
                              osem).wait()

    start_in(0, b0)
    start_in(1, b0 + 1)

    def row(r, carry):
        p = r & 1
        rl = r & 7
        b = b0 + r
        wait_in(p, b)

        @pl.when((rl == 0) & (r >= 8))
        def _():
            wait_out((r >> 3) - 1)

        pvec = jnp.full((16,), p, dtype=jnp.int32)
        for o, src in _TILE_PLAN:
            if src is not None:
                sr, sc = src
                out_v[rl, pl.ds(o, 16)] = in_v[p, sr, pl.ds(sc, 16)]
            else:
                rids = rid_v[pl.ds(o, 16)]
                cids = cid_v[pl.ds(o, 16)]
                out_v[rl, pl.ds(o, 16)] = plsc.load_gather(
                    in_v, [pvec, rids, cids])

        @pl.when(rl == 7)
        def _():
            start_out(r >> 3)

        @pl.when(r + 2 < _BPW)
        def _():
            start_in(p, b + 2)

        return carry

    lax.fori_loop(0, _BPW, row, 0)
    wait_out(_GPW - 1)


def kernel(input):
    rid = jnp.asarray(_RID)
    cid = jnp.asarray(_CID)
    mesh = plsc.VectorSubcoreMesh(core_axis_name="c", subcore_axis_name="s")
    f = pl.kernel(
        _sc_body,
        mesh=mesh,
        out_type=jax.ShapeDtypeStruct((_B, _K), jnp.float32),
        scratch_types=[
            pltpu.VMEM((_K,), jnp.int32),
            pltpu.VMEM((_K,), jnp.int32),
            pltpu.VMEM((2, _N, _N), jnp.float32),
            pltpu.VMEM((8, _K), jnp.float32),
            pltpu.SemaphoreType.DMA,
            pltpu.SemaphoreType.DMA,
        ],
        compiler_params=pltpu.CompilerParams(
            use_tc_tiling_on_sc=True, needs_layout_passes=False
        ),
    )
    return f(input, rid, cid)
